# baseline (speedup 1.0000x reference)
import functools

import jax
import jax.numpy as jnp
from jax import lax
from jax.experimental import pallas as pl
from jax.experimental.pallas import tpu as pltpu
from jax.experimental.pallas import tpu_sc as plsc

_LANES = 128
_SUB = 8
_CHUNK = _LANES * _SUB


def _gather_sc(idx2d, W, n_rows):
    V, D = W.shape
    NC, NS = 2, 16
    NW = NC * NS
    b_per_w = n_rows // NW
    n_chunks = b_per_w // _CHUNK
    mesh = plsc.VectorSubcoreMesh(core_axis_name="c", subcore_axis_name="s")

    @functools.partial(
        pl.kernel,
        mesh=mesh,
        compiler_params=pltpu.CompilerParams(use_tc_tiling_on_sc=False),
        out_type=jax.ShapeDtypeStruct((n_rows, D), jnp.float32),
        scratch_types=[
            pltpu.VMEM((_SUB, _LANES), jnp.int32),
            pltpu.VMEM((_CHUNK, D), jnp.float32),
            pltpu.SemaphoreType.DMA,
        ],
    )
    def k(idx_hbm, w_hbm, out_hbm, idx_v, rows_v, gsem):
        wid = lax.axis_index("s") * NC + lax.axis_index("c")
        base = wid * b_per_w

        def body(i, carry):
            off = pl.multiple_of(base + i * _CHUNK, _CHUNK)
            row = pl.multiple_of(base // _LANES + i * _SUB, _SUB)
            pltpu.sync_copy(idx_hbm.at[pl.ds(row, _SUB)], idx_v)
            copies = [
                pltpu.async_copy(
                    w_hbm.at[idx_v.at[j]],
                    rows_v.at[pl.ds(j * _LANES, _LANES)],
                    gsem,
                )
                for j in range(_SUB)
            ]
            for c in copies:
                c.wait()
            pltpu.sync_copy(rows_v, out_hbm.at[pl.ds(off, _CHUNK)])
            return carry

        lax.fori_loop(0, n_chunks, body, 0)

    return k(idx2d, W)


def kernel(x, W):
    B, H = x.shape
    V, D = W.shape
    N = B * H
    idx2d = x.reshape(N).astype(jnp.int32).reshape(N // _LANES, _LANES)
    out = _gather_sc(idx2d, W, N)
    return out.reshape(B, H, D)
